# baseline (speedup 1.0000x reference)
import functools

import jax
import jax.numpy as jnp
from jax import lax
from jax.experimental import pallas as pl
from jax.experimental.pallas import tpu as pltpu
from jax.experimental.pallas import tpu_sc as plsc

_EMBED = 64
_WORDS = _EMBED // 4
_BATCH_TILE = 128
_SUB = 8


@functools.lru_cache(maxsize=None)
def _build(BATCH, HIST, V):
    info = plsc.get_sparse_core_info()
    NC, NS, L = info.num_cores, info.num_subcores, info.num_lanes
    NW = NC * NS
    assert BATCH % NW == 0
    CHUNK = BATCH // NW
    assert CHUNK % _BATCH_TILE == 0 and CHUNK % L == 0
    n_btiles = CHUNK // _BATCH_TILE
    tile_words = _SUB * _BATCH_TILE
    t_size = _EMBED * CHUNK
    et_seg = n_btiles * tile_words

    mesh = plsc.VectorSubcoreMesh(core_axis_name="c", subcore_axis_name="s")

    @functools.partial(
        pl.kernel,
        mesh=mesh,
        compiler_params=pltpu.CompilerParams(
            needs_layout_passes=False, use_tc_tiling_on_sc=False
        ),
        out_type=jax.ShapeDtypeStruct((BATCH * HIST * _EMBED,), jnp.float32),
        scratch_types=[
            pltpu.VMEM((2, CHUNK), jnp.int32),
            pltpu.VMEM((2, CHUNK, _EMBED), jnp.int8),
            pltpu.VMEM((2, CHUNK), jnp.float32),
            pltpu.VMEM((2, t_size), jnp.float32),
            pltpu.SemaphoreType.DMA,
            pltpu.SemaphoreType.DMA,
            pltpu.SemaphoreType.DMA,
            pltpu.SemaphoreType.DMA,
            pltpu.SemaphoreType.DMA,
            pltpu.SemaphoreType.DMA,
        ],
    )
    def k(idx_hbm, qw_hbm, sc_hbm, out_hbm, idxb, rowsb, sclb, tb,
          semr0, semr1, sems0, sems1, semo0, semo1):
        semr = (semr0, semr1)
        sems = (sems0, sems1)
        semo = (semo0, semo1)
        wid = lax.axis_index("s") * NC + lax.axis_index("c")
        bbase = wid * CHUNK
        ii = lax.iota(jnp.int32, L)
        scat = [
            (ii // 2) * (n_btiles * tile_words)
            + (ii % 2) * (4 * _BATCH_TILE)
            + b2 * _BATCH_TILE
            for b2 in range(4)
        ]

        def stage(h, p):
            pltpu.sync_copy(
                idx_hbm.at[pl.ds(h * BATCH + bbase, CHUNK)], idxb.at[p]
            )
            pltpu.async_copy(qw_hbm.at[idxb.at[p]], rowsb.at[p], semr[p])
            pltpu.async_copy(sc_hbm.at[idxb.at[p]], sclb.at[p], sems[p])

        def wait_gathers(p):
            pltpu.make_async_copy(
                qw_hbm.at[pl.ds(0, CHUNK)], rowsb.at[p], semr[p]
            ).wait()
            pltpu.make_async_copy(
                sc_hbm.at[pl.ds(0, CHUNK)], sclb.at[p], sems[p]
            ).wait()

        def drain_out(p):
            for et in range(_EMBED // _SUB):
                pltpu.make_async_copy(
                    out_hbm.at[pl.ds(0, et_seg)],
                    tb.at[p, pl.ds(et * et_seg, et_seg)],
                    semo[p],
                ).wait()

        def compute(h, p):
            rows = rowsb.at[p]
            tout = tb.at[p]

            def group_body(g, _):
                g16 = g * L
                sv = sclb[p, pl.ds(g16, L)]
                kof_g = (g // (_BATCH_TILE // L)) * tile_words + (
                    g % (_BATCH_TILE // L)
                ) * L
                for i in range(L):
                    r = g16 + i
                    w = plsc.bitcast(rows[r], jnp.int32)
                    s = sv[i]
                    kof = kof_g + i
                    for b2 in range(4):
                        v = (w << (24 - 8 * b2)) >> 24
                        f = v.astype(jnp.float32) * s
                        plsc.store_scatter(tout, [scat[b2] + kof], f)
                return 0

            lax.fori_loop(0, CHUNK // L, group_body, 0)

        def fire_out(h, p):
            for et in range(_EMBED // _SUB):
                base = (
                    (h * _SUB + et) * (BATCH // _BATCH_TILE) + wid * n_btiles
                ) * tile_words
                pltpu.async_copy(
                    tb.at[p, pl.ds(et * et_seg, et_seg)],
                    out_hbm.at[pl.ds(base, et_seg)],
                    semo[p],
                )

        stage(0, 0)

        def h_body(t, _):
            for pp in range(2):
                h = t * 2 + pp
                wait_gathers(pp)

                @pl.when(h + 1 < HIST)
                def _():
                    stage(h + 1, 1 - pp)

                @pl.when(h >= 2)
                def _():
                    drain_out(pp)

                compute(h, pp)
                fire_out(h, pp)
            return 0

        lax.fori_loop(0, HIST // 2, h_body, 0)
        drain_out(0)
        drain_out(1)

    return k


def kernel(input_ids, q_weight, scale):
    BATCH, HIST = input_ids.shape
    V, E = q_weight.shape
    idx_t = input_ids.T.reshape(BATCH * HIST)
    out = _build(BATCH, HIST, V)(idx_t, q_weight, scale.reshape(V))
    out5 = out.reshape(HIST, E // _SUB, BATCH // _BATCH_TILE, _SUB, _BATCH_TILE)
    return out5.transpose(2, 4, 0, 1, 3).reshape(BATCH, HIST, E)


# speedup vs baseline: 1.2787x; 1.2787x over previous
import functools

import jax
import jax.numpy as jnp
from jax import lax
from jax.experimental import pallas as pl
from jax.experimental.pallas import tpu as pltpu
from jax.experimental.pallas import tpu_sc as plsc

_EMBED = 64
_WORDS = _EMBED // 4
_BATCH_TILE = 128
_SUB = 8


@functools.lru_cache(maxsize=None)
def _build(BATCH, HIST, V):
    info = plsc.get_sparse_core_info()
    NC, NS, L = info.num_cores, info.num_subcores, info.num_lanes
    NW = NC * NS
    assert BATCH % NW == 0
    CHUNK = BATCH // NW
    assert CHUNK % _BATCH_TILE == 0 and CHUNK % L == 0
    n_btiles = CHUNK // _BATCH_TILE
    tile_words = _SUB * _BATCH_TILE
    t_size = _EMBED * CHUNK
    et_seg = n_btiles * tile_words

    mesh = plsc.VectorSubcoreMesh(core_axis_name="c", subcore_axis_name="s")

    @functools.partial(
        pl.kernel,
        mesh=mesh,
        compiler_params=pltpu.CompilerParams(
            needs_layout_passes=False, use_tc_tiling_on_sc=False
        ),
        out_type=jax.ShapeDtypeStruct((BATCH * HIST * _EMBED,), jnp.float32),
        scratch_types=[
            pltpu.VMEM((2, CHUNK), jnp.int32),
            pltpu.VMEM((2, CHUNK, _EMBED), jnp.int8),
            pltpu.VMEM((2, CHUNK), jnp.float32),
            pltpu.VMEM((2, t_size), jnp.float32),
            pltpu.SemaphoreType.DMA,
            pltpu.SemaphoreType.DMA,
            pltpu.SemaphoreType.DMA,
            pltpu.SemaphoreType.DMA,
            pltpu.SemaphoreType.DMA,
            pltpu.SemaphoreType.DMA,
        ],
    )
    def k(idx_hbm, qw_hbm, sc_hbm, out_hbm, idxb, rowsb, sclb, tb,
          semr0, semr1, sems0, sems1, semo0, semo1):
        semr = (semr0, semr1)
        sems = (sems0, sems1)
        semo = (semo0, semo1)
        wid = lax.axis_index("s") * NC + lax.axis_index("c")
        bbase = wid * CHUNK
        ii = lax.iota(jnp.int32, L)
        gidx = [4 * j + ii // 4 for j in range(4)]
        shl = (3 - (ii % 4)) * 8

        def stage(h, p):
            pltpu.sync_copy(
                idx_hbm.at[pl.ds(h * BATCH + bbase, CHUNK)], idxb.at[p]
            )
            pltpu.async_copy(qw_hbm.at[idxb.at[p]], rowsb.at[p], semr[p])
            pltpu.async_copy(sc_hbm.at[idxb.at[p]], sclb.at[p], sems[p])

        def wait_gathers(p):
            pltpu.make_async_copy(
                qw_hbm.at[pl.ds(0, CHUNK)], rowsb.at[p], semr[p]
            ).wait()
            pltpu.make_async_copy(
                sc_hbm.at[pl.ds(0, CHUNK)], sclb.at[p], sems[p]
            ).wait()

        def drain_out(p):
            pltpu.make_async_copy(
                out_hbm.at[pl.ds(0, t_size)], tb.at[p], semo[p]
            ).wait()

        def compute(h, p):
            rows = rowsb.at[p]
            tout = tb.at[p]

            def group_body(g, _):
                g16 = g * L
                sv = sclb[p, pl.ds(g16, L)]
                for i in range(L):
                    r = g16 + i
                    w = plsc.bitcast(rows[r], jnp.int32)
                    s = sv[i]
                    for j in range(4):
                        wj = w[gidx[j]]
                        v = (wj << shl) >> 24
                        f = v.astype(jnp.float32) * s
                        tout[pl.ds(r * _EMBED + j * L, L)] = f
                return 0

            lax.fori_loop(0, CHUNK // L, group_body, 0)

        def fire_out(h, p):
            pltpu.async_copy(
                tb.at[p],
                out_hbm.at[pl.ds((h * BATCH + bbase) * _EMBED, t_size)],
                semo[p],
            )

        stage(0, 0)

        def h_body(t, _):
            for pp in range(2):
                h = t * 2 + pp
                wait_gathers(pp)

                @pl.when(h + 1 < HIST)
                def _():
                    stage(h + 1, 1 - pp)

                @pl.when(h >= 2)
                def _():
                    drain_out(pp)

                compute(h, pp)
                fire_out(h, pp)
            return 0

        lax.fori_loop(0, HIST // 2, h_body, 0)
        drain_out(0)
        drain_out(1)

    return k


def kernel(input_ids, q_weight, scale):
    BATCH, HIST = input_ids.shape
    V, E = q_weight.shape
    idx_t = input_ids.T.reshape(BATCH * HIST)
    out = _build(BATCH, HIST, V)(idx_t, q_weight, scale.reshape(V))
    return out.reshape(HIST, BATCH, E).transpose(1, 0, 2)


# speedup vs baseline: 1.7832x; 1.3945x over previous
import functools

import jax
import jax.numpy as jnp
from jax import lax
from jax.experimental import pallas as pl
from jax.experimental.pallas import tpu as pltpu
from jax.experimental.pallas import tpu_sc as plsc

_EMBED = 64
_WORDS = _EMBED // 4
_BATCH_TILE = 128
_SUB = 8


@functools.lru_cache(maxsize=None)
def _build(BATCH, HIST, V):
    info = plsc.get_sparse_core_info()
    NC, NS, L = info.num_cores, info.num_subcores, info.num_lanes
    NW = NC * NS
    assert BATCH % NW == 0
    CHUNK = BATCH // NW
    assert CHUNK % _BATCH_TILE == 0 and CHUNK % L == 0
    n_btiles = CHUNK // _BATCH_TILE
    tile_words = _SUB * _BATCH_TILE
    t_size = _EMBED * CHUNK
    et_seg = n_btiles * tile_words

    mesh = plsc.VectorSubcoreMesh(core_axis_name="c", subcore_axis_name="s")

    @functools.partial(
        pl.kernel,
        mesh=mesh,
        compiler_params=pltpu.CompilerParams(
            needs_layout_passes=False, use_tc_tiling_on_sc=False
        ),
        out_type=jax.ShapeDtypeStruct((BATCH * HIST * _EMBED,), jnp.float32),
        scratch_types=[
            pltpu.VMEM((2, CHUNK), jnp.int32),
            pltpu.VMEM((2, CHUNK, _EMBED), jnp.int8),
            pltpu.VMEM((2, CHUNK), jnp.float32),
            pltpu.VMEM((2, t_size), jnp.float32),
            pltpu.VMEM((CHUNK * (_WORDS + 1),), jnp.int32),
            pltpu.SemaphoreType.DMA,
            pltpu.SemaphoreType.DMA,
            pltpu.SemaphoreType.DMA,
            pltpu.SemaphoreType.DMA,
            pltpu.SemaphoreType.DMA,
            pltpu.SemaphoreType.DMA,
        ],
    )
    def k(idx_hbm, qw_hbm, sc_hbm, out_hbm, idxb, rowsb, sclb, tb, wp,
          semr0, semr1, sems0, sems1, semo0, semo1):
        semr = (semr0, semr1)
        sems = (sems0, sems1)
        semo = (semo0, semo1)
        wid = lax.axis_index("s") * NC + lax.axis_index("c")
        bbase = wid * CHUNK
        ii = lax.iota(jnp.int32, L)
        PITCH = _WORDS + 1

        def stage(h, p):
            pltpu.sync_copy(
                idx_hbm.at[pl.ds(h * BATCH + bbase, CHUNK)], idxb.at[p]
            )
            pltpu.async_copy(qw_hbm.at[idxb.at[p]], rowsb.at[p], semr[p])
            pltpu.async_copy(sc_hbm.at[idxb.at[p]], sclb.at[p], sems[p])

        def wait_gathers(p):
            pltpu.make_async_copy(
                qw_hbm.at[pl.ds(0, CHUNK)], rowsb.at[p], semr[p]
            ).wait()
            pltpu.make_async_copy(
                sc_hbm.at[pl.ds(0, CHUNK)], sclb.at[p], sems[p]
            ).wait()

        def drain_out(p):
            for et in range(_EMBED // _SUB):
                pltpu.make_async_copy(
                    out_hbm.at[pl.ds(0, et_seg)],
                    tb.at[p, pl.ds(et * et_seg, et_seg)],
                    semo[p],
                ).wait()

        def compute(h, p):
            rows = rowsb.at[p]
            tout = tb.at[p]

            def pad_body(g, _):
                g16 = g * L
                for i in range(L):
                    r = g16 + i
                    wp[pl.ds(r * PITCH, _WORDS)] = plsc.bitcast(
                        rows[r], jnp.int32
                    )
                return 0

            lax.fori_loop(0, CHUNK // L, pad_body, 0)

            def group_body(kg, _):
                sv = sclb[p, pl.ds(kg * L, L)]
                kbase = (kg * L + ii) * PITCH
                kgoff = (kg // (_BATCH_TILE // L)) * tile_words + (
                    kg % (_BATCH_TILE // L)
                ) * L
                for wi in range(_WORDS):
                    w = plsc.load_gather(wp, [kbase + wi])
                    for b2 in range(4):
                        e = wi * 4 + b2
                        v = (w << (24 - 8 * b2)) >> 24
                        f = v.astype(jnp.float32) * sv
                        taddr = (e // _SUB) * et_seg + (e % _SUB) * _BATCH_TILE
                        tout[pl.ds(kgoff + taddr, L)] = f
                return 0

            lax.fori_loop(0, CHUNK // L, group_body, 0)

        def fire_out(h, p):
            for et in range(_EMBED // _SUB):
                base = (
                    (h * _SUB + et) * (BATCH // _BATCH_TILE) + wid * n_btiles
                ) * tile_words
                pltpu.async_copy(
                    tb.at[p, pl.ds(et * et_seg, et_seg)],
                    out_hbm.at[pl.ds(base, et_seg)],
                    semo[p],
                )

        stage(0, 0)

        def h_body(t, _):
            for pp in range(2):
                h = t * 2 + pp
                wait_gathers(pp)

                @pl.when(h + 1 < HIST)
                def _():
                    stage(h + 1, 1 - pp)

                @pl.when(h >= 2)
                def _():
                    drain_out(pp)

                compute(h, pp)
                fire_out(h, pp)
            return 0

        lax.fori_loop(0, HIST // 2, h_body, 0)
        drain_out(0)
        drain_out(1)

    return k


def kernel(input_ids, q_weight, scale):
    BATCH, HIST = input_ids.shape
    V, E = q_weight.shape
    idx_t = input_ids.T.reshape(BATCH * HIST)
    out = _build(BATCH, HIST, V)(idx_t, q_weight, scale.reshape(V))
    out5 = out.reshape(HIST, E // _SUB, BATCH // _BATCH_TILE, _SUB, _BATCH_TILE)
    return out5.transpose(2, 4, 0, 1, 3).reshape(BATCH, HIST, E)
